# baseline (speedup 1.0000x reference)
import jax
import jax.numpy as jnp
import numpy as np
from jax.experimental import pallas as pl
from jax.experimental.pallas import tpu as pltpu

_PRED_FREQ = np.array([712432, 253342, 208287, 197550, 66425, 47342, 33637,
                       32347, 21575, 15457, 13715, 13360, 10191, 9903, 9894,
                       9317, 9145, 8856, 6712, 5213, 4688, 4613, 4507, 4465,
                       4021, 3810, 3806, 3739, 3624, 3490, 3477, 3411, 3288,
                       3095, 3092, 3083, 2945, 2721, 2517, 2450, 2312, 2253,
                       2241, 2065, 1829, 1603, 1413, 1225, 793, 712, 663],
                      dtype=np.float32)
_GAMMA = 2.0
_ALPHA = 0.4
_TAU = 1.0
_FG_BOOST = 1.2
_N = 262144
_C = 51

_BN = 65536


def _log_priors() -> np.ndarray:
    pf = np.nan_to_num(_PRED_FREQ, nan=1e-06)
    pf = np.clip(pf, 1e-12, None)
    priors = pf / (pf.sum(dtype=np.float32) + 1e-12)
    return np.clip(np.log(priors + 1e-12), -20.0, 20.0).astype(np.float32)


def _body(x_ref, tp_ref, lp_ref, loss_ref, wsum_ref):
    i = pl.program_id(0)
    xt = x_ref[...]
    tp = tp_ref[0]
    lp = lp_ref[...]
    adj = jnp.clip(xt + lp, -50.0, 50.0)
    e = jnp.exp(adj)
    cls = jax.lax.broadcasted_iota(jnp.int32, (_C, _BN), 0)
    ev = jnp.sum(jnp.where(cls == tp, e, 0.0), axis=0, keepdims=True)
    s = jnp.sum(e, axis=0, keepdims=True)
    ce = jnp.log(s) - jnp.log(ev)
    pt = ev / s
    fw = (1.0 - pt) * (1.0 - pt)
    w = jnp.where(tp == 0, jnp.float32(_ALPHA),
                  jnp.float32((1.0 - _ALPHA) * _FG_BOOST))
    part_loss = jnp.sum(ce * fw * w)
    part_w = jnp.sum(w)

    @pl.when(i == 0)
    def _():
        loss_ref[0, 0] = 0.0
        wsum_ref[0, 0] = 0.0

    loss_ref[0, 0] += part_loss
    wsum_ref[0, 0] += part_w


def kernel(logits, target):
    xt = jnp.transpose(logits.astype(jnp.float32))
    t_pack = target.astype(jnp.int32).reshape(_N // _BN, 1, _BN)
    lp = jnp.asarray(_log_priors()).reshape(_C, 1)
    grid = _N // _BN
    loss_sum, w_sum = pl.pallas_call(
        _body,
        grid=(grid,),
        in_specs=[
            pl.BlockSpec((_C, _BN), lambda i: (0, i)),
            pl.BlockSpec((1, 1, _BN), lambda i: (i, 0, 0)),
            pl.BlockSpec((_C, 1), lambda i: (0, 0)),
        ],
        out_specs=[
            pl.BlockSpec((1, 1), lambda i: (0, 0), memory_space=pltpu.SMEM),
            pl.BlockSpec((1, 1), lambda i: (0, 0), memory_space=pltpu.SMEM),
        ],
        out_shape=[
            jax.ShapeDtypeStruct((1, 1), jnp.float32),
            jax.ShapeDtypeStruct((1, 1), jnp.float32),
        ],
    )(xt, t_pack, lp)
    normalizer = jnp.clip(w_sum[0, 0], 1.0, None)
    return loss_sum[0, 0] / normalizer


# speedup vs baseline: 1.1192x; 1.1192x over previous
import jax
import jax.numpy as jnp
import numpy as np
from jax.experimental import pallas as pl
from jax.experimental.pallas import tpu as pltpu

_PRED_FREQ = np.array([712432, 253342, 208287, 197550, 66425, 47342, 33637,
                       32347, 21575, 15457, 13715, 13360, 10191, 9903, 9894,
                       9317, 9145, 8856, 6712, 5213, 4688, 4613, 4507, 4465,
                       4021, 3810, 3806, 3739, 3624, 3490, 3477, 3411, 3288,
                       3095, 3092, 3083, 2945, 2721, 2517, 2450, 2312, 2253,
                       2241, 2065, 1829, 1603, 1413, 1225, 793, 712, 663],
                      dtype=np.float32)
_GAMMA = 2.0
_ALPHA = 0.4
_TAU = 1.0
_FG_BOOST = 1.2
_N = 262144
_C = 51

_BN = 32768


def _log_priors() -> np.ndarray:
    pf = np.nan_to_num(_PRED_FREQ, nan=1e-06)
    pf = np.clip(pf, 1e-12, None)
    priors = pf / (pf.sum(dtype=np.float32) + 1e-12)
    return np.clip(np.log(priors + 1e-12), -20.0, 20.0).astype(np.float32)


def _body(x_ref, tp_ref, lp_ref, loss_ref, wsum_ref):
    i = pl.program_id(0)
    xt = x_ref[...]
    tp = tp_ref[0]
    lp = lp_ref[...]
    adj = jnp.clip(xt + lp, -50.0, 50.0)
    e = jnp.exp(adj)
    cls = jax.lax.broadcasted_iota(jnp.int32, (_C, _BN), 0)
    ev = jnp.sum(jnp.where(cls == tp, e, 0.0), axis=0, keepdims=True)
    s = jnp.sum(e, axis=0, keepdims=True)
    ce = jnp.log(s) - jnp.log(ev)
    pt = ev / s
    fw = (1.0 - pt) * (1.0 - pt)
    w = jnp.where(tp == 0, jnp.float32(_ALPHA),
                  jnp.float32((1.0 - _ALPHA) * _FG_BOOST))
    part_loss = jnp.sum(ce * fw * w)
    part_w = jnp.sum(w)

    @pl.when(i == 0)
    def _():
        loss_ref[0, 0] = 0.0
        wsum_ref[0, 0] = 0.0

    loss_ref[0, 0] += part_loss
    wsum_ref[0, 0] += part_w


def kernel(logits, target):
    xt = jnp.transpose(logits.astype(jnp.float32))
    t_pack = target.astype(jnp.int32).reshape(_N // _BN, 1, _BN)
    lp = jnp.asarray(_log_priors()).reshape(_C, 1)
    grid = _N // _BN
    loss_sum, w_sum = pl.pallas_call(
        _body,
        grid=(grid,),
        in_specs=[
            pl.BlockSpec((_C, _BN), lambda i: (0, i)),
            pl.BlockSpec((1, 1, _BN), lambda i: (i, 0, 0)),
            pl.BlockSpec((_C, 1), lambda i: (0, 0)),
        ],
        out_specs=[
            pl.BlockSpec((1, 1), lambda i: (0, 0), memory_space=pltpu.SMEM),
            pl.BlockSpec((1, 1), lambda i: (0, 0), memory_space=pltpu.SMEM),
        ],
        out_shape=[
            jax.ShapeDtypeStruct((1, 1), jnp.float32),
            jax.ShapeDtypeStruct((1, 1), jnp.float32),
        ],
    )(xt, t_pack, lp)
    normalizer = jnp.clip(w_sum[0, 0], 1.0, None)
    return loss_sum[0, 0] / normalizer
